# baseline (speedup 1.0000x reference)
import functools

import jax
import jax.numpy as jnp
from jax import lax
from jax.experimental import pallas as pl
from jax.experimental.pallas import tpu as pltpu
from jax.experimental.pallas import tpu_sc as plsc

_N = 2048
_B = 8
_R = _B * _N
_NW = 32
_RPW = _R // _NW


def _bf16_round(v):
    u = lax.bitcast_convert_type(v, jnp.int32)
    r = (u + jnp.int32(0x7FFF)
         + (lax.shift_right_logical(u, jnp.int32(16)) & jnp.int32(1)))
    r = r & jnp.int32(-65536)
    return lax.bitcast_convert_type(r, jnp.float32)


def _perm(x, idx):
    return lax.gather(
        x, idx[:, None],
        lax.GatherDimensionNumbers(
            offset_dims=(), collapsed_slice_dims=(0,), start_index_map=(0,)),
        (1,), mode=lax.GatherScatterMode.PROMISE_IN_BOUNDS)


def _top16(drow, base, cand_v, cand_i):
    iota = lax.iota(jnp.int32, 16)
    inf16 = jnp.full((16,), jnp.inf, jnp.float32)

    def p1(c, ms):
        b = base + c * 64
        return tuple(jnp.minimum(ms[k], drow[pl.ds(b + k * 16, 16)])
                     for k in range(4))
    ms = lax.fori_loop(0, 32, p1, (inf16, inf16, inf16, inf16), unroll=4)
    m = jnp.minimum(jnp.minimum(ms[0], ms[1]), jnp.minimum(ms[2], ms[3]))
    tau = jnp.max(m)

    def p2(c, off):
        v = drow[pl.ds(base + c * 16, 16)]
        msk = v <= tau
        plsc.store_compressed(cand_v.at[pl.ds(off, 16)], v, mask=msk)
        plsc.store_compressed(cand_i.at[pl.ds(off, 16)], iota + c * 16,
                              mask=msk)
        return off + jnp.sum(msk.astype(jnp.int32))
    off = lax.fori_loop(0, 128, p2, jnp.int32(0), unroll=4)

    cand_v[pl.ds(off, 16)] = inf16
    cand_i[pl.ds(off, 16)] = jnp.zeros((16,), jnp.int32)

    rv, ri = plsc.sort_key_val(cand_v[pl.ds(0, 16)], cand_i[pl.ds(0, 16)])
    nch = (off + 15) >> 4

    def p3(c, carry):
        rv, ri = carry
        sv, si = plsc.sort_key_val(cand_v[pl.ds(c * 16, 16)],
                                   cand_i[pl.ds(c * 16, 16)])
        bv = lax.rev(sv, (0,))
        bi = lax.rev(si, (0,))
        ta = (rv < bv) | ((rv == bv) & (ri < bi))
        lv = jnp.where(ta, rv, bv)
        li = jnp.where(ta, ri, bi)
        for s in (8, 4, 2, 1):
            pv = _perm(lv, jnp.bitwise_xor(iota, s))
            pi = _perm(li, jnp.bitwise_xor(iota, s))
            amin = (lv < pv) | ((lv == pv) & (li < pi))
            minv = jnp.where(amin, lv, pv)
            mini = jnp.where(amin, li, pi)
            maxv = jnp.where(amin, pv, lv)
            maxi = jnp.where(amin, pi, li)
            low = (iota & s) == 0
            lv = jnp.where(low, minv, maxv)
            li = jnp.where(low, mini, maxi)
        return lv, li

    rv, ri = lax.fori_loop(1, nch, p3, (rv, ri))
    return ri


def _make_sc_knn(mode, out_c):
    mesh = plsc.VectorSubcoreMesh(core_axis_name="c", subcore_axis_name="s",
                                  num_cores=2, num_subcores=16)

    @functools.partial(
        pl.kernel, mesh=mesh,
        compiler_params=pltpu.CompilerParams(needs_layout_passes=False),
        out_type=jax.ShapeDtypeStruct((_R, 128), jnp.float32),
        scratch_types=[
            pltpu.VMEM((2 * _N,), jnp.float32),
            pltpu.VMEM((2080,), jnp.float32),
            pltpu.VMEM((2080,), jnp.int32),
            pltpu.VMEM((16, 128), jnp.float32),
            pltpu.VMEM((2 * 128,), jnp.float32),
            pltpu.SemaphoreType.DMA,
            pltpu.SemaphoreType.DMA,
            pltpu.SemaphoreType.DMA,
        ])
    def knn_kernel(d_hbm, f_hbm, out_hbm, drow, cand_v, cand_i, gbuf,
                   orow, rsem, gsem, osem):
        wid = lax.axis_index("s") * 2 + lax.axis_index("c")
        r0 = wid * _RPW
        iota = lax.iota(jnp.int32, 16)

        for z in range(2 * 128 // 16):
            if (z * 16) % 128 >= out_c:
                orow[pl.ds(z * 16, 16)] = jnp.zeros((16,), jnp.float32)

        pltpu.async_copy(d_hbm.at[r0], drow.at[pl.ds(0, _N)], rsem)

        def row_body(lr, _):
            cur = lr & 1
            r = r0 + lr
            pltpu.make_async_copy(d_hbm.at[r], drow.at[pl.ds(cur * _N, _N)],
                                  rsem).wait()

            @pl.when(lr + 1 < _RPW)
            def _():
                pltpu.async_copy(d_hbm.at[r + 1],
                                 drow.at[pl.ds((1 - cur) * _N, _N)], rsem)

            ri = _top16(drow, cur * _N, cand_v, cand_i)
            gidx = ri + ((r >> 11) << 11)
            pltpu.async_copy(f_hbm.at[gidx], gbuf, gsem).wait()

            @pl.when(lr >= 2)
            def _():
                pltpu.make_async_copy(orow.at[pl.ds(cur * 128, 128)],
                                      out_hbm.at[r - 2], osem).wait()

            ob = cur * 128
            if mode == 'cov':
                xs = [plsc.load_gather(gbuf, [iota, jnp.full((16,), c,
                                                             jnp.int32)])
                      for c in range(3)]
                mus = [jnp.sum(v) * jnp.float32(0.0625) for v in xs]
                ctr = [_bf16_round(v - mu) for v, mu in zip(xs, mus)]
                row = jnp.zeros((16,), jnp.float32)
                for a in range(3):
                    for b in range(a, 3):
                        cab = jnp.sum(ctr[a] * ctr[b])
                        row = jnp.where(iota == 3 * a + b,
                                        jnp.full((16,), cab), row)
                        if a != b:
                            row = jnp.where(iota == 3 * b + a,
                                            jnp.full((16,), cab), row)
                orow[pl.ds(ob, 16)] = row
            else:
                for cc in range(out_c // 16):
                    sl = pl.ds(cc * 16, 16)
                    a0 = jnp.maximum(gbuf[0, sl], gbuf[1, sl])
                    a1 = jnp.maximum(gbuf[2, sl], gbuf[3, sl])
                    a2 = jnp.maximum(gbuf[4, sl], gbuf[5, sl])
                    a3 = jnp.maximum(gbuf[6, sl], gbuf[7, sl])
                    a4 = jnp.maximum(gbuf[8, sl], gbuf[9, sl])
                    a5 = jnp.maximum(gbuf[10, sl], gbuf[11, sl])
                    a6 = jnp.maximum(gbuf[12, sl], gbuf[13, sl])
                    a7 = jnp.maximum(gbuf[14, sl], gbuf[15, sl])
                    acc = jnp.maximum(
                        jnp.maximum(jnp.maximum(a0, a1), jnp.maximum(a2, a3)),
                        jnp.maximum(jnp.maximum(a4, a5), jnp.maximum(a6, a7)))
                    orow[pl.ds(ob + cc * 16, 16)] = acc
            pltpu.async_copy(orow.at[pl.ds(ob, 128)], out_hbm.at[r], osem)
            return 0

        lax.fori_loop(0, _RPW, row_body, 0)
        pltpu.make_async_copy(orow.at[pl.ds(0, 128)],
                              out_hbm.at[r0], osem).wait()
        pltpu.make_async_copy(orow.at[pl.ds(128, 128)],
                              out_hbm.at[r0], osem).wait()

    return knn_kernel


def _mm(a, b):
    return lax.dot_general(a.astype(jnp.bfloat16), b.astype(jnp.bfloat16),
                           (((1,), (0,)), ((), ())),
                           preferred_element_type=jnp.float32)


def _dist_kernel(xr_ref, xct_ref, o_ref):
    xr = xr_ref[...]
    xct = xct_ref[0]
    sr = jnp.sum(xr * xr, axis=1, keepdims=True)
    sc = jnp.sum(xct * xct, axis=0, keepdims=True)
    o_ref[...] = (sr - 2.0 * _mm(xr, xct)) + sc


def _dist(f, ft, c):
    rb = 256
    nb = _N // rb
    return pl.pallas_call(
        _dist_kernel,
        grid=(_B, nb),
        in_specs=[
            pl.BlockSpec((rb, c), lambda b, i: (b * nb + i, 0)),
            pl.BlockSpec((1, c, _N), lambda b, i: (b, 0, 0)),
        ],
        out_specs=pl.BlockSpec((rb, _N), lambda b, i: (b * nb + i, 0)),
        out_shape=jax.ShapeDtypeStruct((_R, _N), jnp.float32),
    )(f, ft)


def _mlp1_kernel(x_ref, cov_ref, w1a_ref, w1b_ref, b1_ref, s1_ref, e1_ref,
                 w2_ref, b2_ref, s2_ref, e2_ref,
                 w3_ref, b3_ref, s3_ref, e3_ref, o_ref):
    h = (_mm(x_ref[...], w1a_ref[...]) + _mm(cov_ref[...], w1b_ref[...])
         + b1_ref[...])
    h = jnp.maximum(h * s1_ref[...] + e1_ref[...], 0.0)
    h = jnp.maximum((_mm(h, w2_ref[...]) + b2_ref[...]) * s2_ref[...]
                    + e2_ref[...], 0.0)
    h = jnp.maximum((_mm(h, w3_ref[...]) + b3_ref[...]) * s3_ref[...]
                    + e3_ref[...], 0.0)
    o_ref[...] = h


def _mlp2_kernel(p_ref, wl_ref, bl_ref, wc_ref, bc_ref, o_ref):
    h = _mm(p_ref[...], wl_ref[...]) + bl_ref[...]
    o_ref[...] = jnp.maximum(_mm(h, wc_ref[...]) + bc_ref[...], 0.0)


def _mlp3_kernel(p_ref, wl_ref, bl_ref, wc_ref, bc_ref, o_ref):
    i = pl.program_id(1)
    h = _mm(p_ref[...], wl_ref[...]) + bl_ref[...]
    g = _mm(h, wc_ref[...]) + bc_ref[...]
    m = jnp.max(g, axis=0, keepdims=True)[None]

    @pl.when(i == 0)
    def _():
        o_ref[...] = m

    @pl.when(i > 0)
    def _():
        o_ref[...] = jnp.maximum(o_ref[...], m)


def _head_kernel(x_ref, w4_ref, b4_ref, w5_ref, b5_ref, o_ref):
    h = jnp.maximum(_mm(x_ref[...], w4_ref[...]) + b4_ref[...], 0.0)
    o_ref[...] = _mm(h, w5_ref[...]) + b5_ref[...]


def _rows_spec(rb, nc):
    return pl.BlockSpec((rb, nc), lambda i: (i, 0))


def _full(shape):
    nd = len(shape)
    return pl.BlockSpec(shape, lambda *a: (0,) * nd)


@functools.lru_cache(maxsize=None)
def _sc_kernels():
    return (_make_sc_knn('cov', 16),
            _make_sc_knn('max', 64),
            _make_sc_knn('max', 128))


def kernel(x, params):
    p = params
    sc_cov, sc_pool64, sc_pool128 = _sc_kernels()

    xflat = x.reshape(_R, 3)
    xpad = jnp.concatenate(
        [xflat, jnp.zeros((_R, 125), jnp.float32)], axis=1)
    xpad_t = xpad.reshape(_B, _N, 128).transpose(0, 2, 1)

    d1 = _dist(xpad, xpad_t, 128)
    cov16 = sc_cov(d1, xpad)

    x8 = jnp.concatenate([xflat, jnp.zeros((_R, 5), jnp.float32)], axis=1)
    w1a8 = jnp.concatenate(
        [p['W1'][:3], jnp.zeros((5, 12), jnp.float32)], axis=0)
    w1b128 = jnp.concatenate(
        [p['W1'][3:], jnp.zeros((119, 12), jnp.float32)], axis=0)

    zc = jnp.zeros((64,), jnp.float32)
    w3p = jnp.concatenate([p['W3'], jnp.zeros((64, 64), jnp.float32)], axis=1)
    b3p = jnp.concatenate([p['b3'], zc])
    g3p = jnp.concatenate([p['g3'], zc])
    e3p = jnp.concatenate([p['be3'], zc])

    bnc = jnp.sqrt(jnp.float32(1.0 + 1e-3))
    s1 = p['g1'] / bnc
    s2 = p['g2'] / bnc
    s3 = g3p / bnc

    rb = 512
    nb = _R // rb
    f1 = pl.pallas_call(
        _mlp1_kernel,
        grid=(nb,),
        in_specs=[
            _rows_spec(rb, 8), _rows_spec(rb, 128),
            _full((8, 12)), _full((128, 12)),
            _full((1, 12)), _full((1, 12)), _full((1, 12)),
            _full((12, 64)), _full((1, 64)), _full((1, 64)), _full((1, 64)),
            _full((64, 128)), _full((1, 128)), _full((1, 128)),
            _full((1, 128)),
        ],
        out_specs=_rows_spec(rb, 128),
        out_shape=jax.ShapeDtypeStruct((_R, 128), jnp.float32),
    )(x8, cov16, w1a8, w1b128,
      p['b1'][None, :], s1[None, :], p['be1'][None, :],
      p['W2'], p['b2'][None, :], s2[None, :], p['be2'][None, :],
      w3p, b3p[None, :], s3[None, :], e3p[None, :])

    f1t = f1.reshape(_B, _N, 128).transpose(0, 2, 1)
    d2 = _dist(f1, f1t, 128)
    p2 = sc_pool64(d2, f1)

    wl1p = jnp.concatenate(
        [p['Wl1'], jnp.zeros((64, 64), jnp.float32)], axis=0)
    f2 = pl.pallas_call(
        _mlp2_kernel,
        grid=(nb,),
        in_specs=[
            _rows_spec(rb, 128),
            _full((128, 64)), _full((1, 64)),
            _full((64, 128)), _full((1, 128)),
        ],
        out_specs=_rows_spec(rb, 128),
        out_shape=jax.ShapeDtypeStruct((_R, 128), jnp.float32),
    )(p2, wl1p, p['bl1'][None, :], p['Wc1'], p['bc1'][None, :])

    f2t = f2.reshape(_B, _N, 128).transpose(0, 2, 1)
    d3 = _dist(f2, f2t, 128)
    p3 = sc_pool128(d3, f2)

    nbb = _N // rb
    m = pl.pallas_call(
        _mlp3_kernel,
        grid=(_B, nbb),
        in_specs=[
            pl.BlockSpec((rb, 128), lambda b, i: (b * nbb + i, 0)),
            _full((128, 128)), _full((1, 128)),
            _full((128, 1024)), _full((1, 1024)),
        ],
        out_specs=pl.BlockSpec((1, 1, 1024), lambda b, i: (b, 0, 0)),
        out_shape=jax.ShapeDtypeStruct((_B, 1, 1024), jnp.float32),
    )(p3, p['Wl2'], p['bl2'][None, :], p['Wc2'], p['bc2'][None, :])
    m = m.reshape(_B, 1024)

    out = pl.pallas_call(
        _head_kernel,
        in_specs=[
            _full((_B, 1024)),
            _full((1024, 1024)), _full((1, 1024)),
            _full((1024, 512)), _full((1, 512)),
        ],
        out_specs=_full((_B, 512)),
        out_shape=jax.ShapeDtypeStruct((_B, 512), jnp.float32),
    )(m, p['W4'], p['b4'][None, :], p['W5'], p['b5'][None, :])

    return out[:, None, :]


# speedup vs baseline: 1.3430x; 1.3430x over previous
import functools

import jax
import jax.numpy as jnp
from jax import lax
from jax.experimental import pallas as pl
from jax.experimental.pallas import tpu as pltpu
from jax.experimental.pallas import tpu_sc as plsc

_N = 2048
_B = 8
_R = _B * _N
_NW = 32
_RPW = _R // _NW


def _bf16_round(v):
    u = lax.bitcast_convert_type(v, jnp.int32)
    r = (u + jnp.int32(0x7FFF)
         + (lax.shift_right_logical(u, jnp.int32(16)) & jnp.int32(1)))
    r = r & jnp.int32(-65536)
    return lax.bitcast_convert_type(r, jnp.float32)


def _perm(x, idx):
    return lax.gather(
        x, idx[:, None],
        lax.GatherDimensionNumbers(
            offset_dims=(), collapsed_slice_dims=(0,), start_index_map=(0,)),
        (1,), mode=lax.GatherScatterMode.PROMISE_IN_BOUNDS)


def _top16(drow, base, cand_v, cand_i):
    iota = lax.iota(jnp.int32, 16)
    inf16 = jnp.full((16,), jnp.inf, jnp.float32)

    def p1(c, ms):
        b = base + c * 64
        return tuple(jnp.minimum(ms[k], drow[pl.ds(b + k * 16, 16)])
                     for k in range(4))
    ms = lax.fori_loop(0, 32, p1, (inf16, inf16, inf16, inf16), unroll=4)
    m = jnp.minimum(jnp.minimum(ms[0], ms[1]), jnp.minimum(ms[2], ms[3]))
    tau = jnp.max(m)

    def p2(c, off):
        v = drow[pl.ds(base + c * 16, 16)]
        msk = v <= tau
        plsc.store_compressed(cand_v.at[pl.ds(off, 16)], v, mask=msk)
        plsc.store_compressed(cand_i.at[pl.ds(off, 16)], iota + c * 16,
                              mask=msk)
        return off + jnp.sum(msk.astype(jnp.int32))
    off = lax.fori_loop(0, 128, p2, jnp.int32(0), unroll=4)

    cand_v[pl.ds(off, 16)] = inf16
    cand_i[pl.ds(off, 16)] = jnp.zeros((16,), jnp.int32)

    rv, ri = plsc.sort_key_val(cand_v[pl.ds(0, 16)], cand_i[pl.ds(0, 16)])
    nch = (off + 15) >> 4

    def p3(c, carry):
        rv, ri = carry
        sv, si = plsc.sort_key_val(cand_v[pl.ds(c * 16, 16)],
                                   cand_i[pl.ds(c * 16, 16)])
        bv = lax.rev(sv, (0,))
        bi = lax.rev(si, (0,))
        ta = (rv < bv) | ((rv == bv) & (ri < bi))
        lv = jnp.where(ta, rv, bv)
        li = jnp.where(ta, ri, bi)
        for s in (8, 4, 2, 1):
            pv = _perm(lv, jnp.bitwise_xor(iota, s))
            pi = _perm(li, jnp.bitwise_xor(iota, s))
            amin = (lv < pv) | ((lv == pv) & (li < pi))
            minv = jnp.where(amin, lv, pv)
            mini = jnp.where(amin, li, pi)
            maxv = jnp.where(amin, pv, lv)
            maxi = jnp.where(amin, pi, li)
            low = (iota & s) == 0
            lv = jnp.where(low, minv, maxv)
            li = jnp.where(low, mini, maxi)
        return lv, li

    rv, ri = lax.fori_loop(1, nch, p3, (rv, ri))
    return ri


def _make_sc_knn(mode, out_c):
    mesh = plsc.VectorSubcoreMesh(core_axis_name="c", subcore_axis_name="s",
                                  num_cores=2, num_subcores=16)

    def _reduce(gbuf, boff, orow, ob, iota):
        if mode == 'cov':
            xs = [plsc.load_gather(gbuf, [iota + boff,
                                          jnp.full((16,), c, jnp.int32)])
                  for c in range(3)]
            mus = [jnp.sum(v) * jnp.float32(0.0625) for v in xs]
            ctr = [_bf16_round(v - mu) for v, mu in zip(xs, mus)]
            row = jnp.zeros((16,), jnp.float32)
            for a in range(3):
                for b in range(a, 3):
                    cab = jnp.sum(ctr[a] * ctr[b])
                    row = jnp.where(iota == 3 * a + b,
                                    jnp.full((16,), cab), row)
                    if a != b:
                        row = jnp.where(iota == 3 * b + a,
                                        jnp.full((16,), cab), row)
            orow[pl.ds(ob, 16)] = row
        else:
            for cc in range(out_c // 16):
                sl = pl.ds(cc * 16, 16)
                accs = [jnp.maximum(gbuf[boff + 2 * j, sl],
                                    gbuf[boff + 2 * j + 1, sl])
                        for j in range(8)]
                acc = jnp.maximum(
                    jnp.maximum(jnp.maximum(accs[0], accs[1]),
                                jnp.maximum(accs[2], accs[3])),
                    jnp.maximum(jnp.maximum(accs[4], accs[5]),
                                jnp.maximum(accs[6], accs[7])))
                orow[pl.ds(ob + cc * 16, 16)] = acc

    @functools.partial(
        pl.kernel, mesh=mesh,
        compiler_params=pltpu.CompilerParams(needs_layout_passes=False),
        out_type=jax.ShapeDtypeStruct((_R, 128), jnp.float32),
        scratch_types=[
            pltpu.VMEM((2 * _N,), jnp.float32),
            pltpu.VMEM((2080,), jnp.float32),
            pltpu.VMEM((2080,), jnp.int32),
            pltpu.VMEM((32, 128), jnp.float32),
            pltpu.VMEM((2 * 128,), jnp.float32),
            pltpu.SemaphoreType.DMA,
            pltpu.SemaphoreType.DMA,
            pltpu.SemaphoreType.DMA,
            pltpu.SemaphoreType.DMA,
            pltpu.SemaphoreType.DMA,
        ])
    def knn_kernel(d_hbm, f_hbm, out_hbm, drow, cand_v, cand_i, gbuf,
                   orow, rsem, gsema, gsemb, osema, osemb):
        wid = lax.axis_index("s") * 2 + lax.axis_index("c")
        r0 = wid * _RPW
        bbase = jnp.full((16,), (r0 >> 11) << 11, jnp.int32)
        iota = lax.iota(jnp.int32, 16)

        for z in range(2 * 128 // 16):
            if (z * 16) % 128 >= out_c:
                orow[pl.ds(z * 16, 16)] = jnp.zeros((16,), jnp.float32)

        pltpu.async_copy(d_hbm.at[r0], drow.at[pl.ds(0, _N)], rsem)

        def iter_body(i, _):
            r = r0 + 2 * i
            pltpu.make_async_copy(d_hbm.at[r], drow.at[pl.ds(0, _N)],
                                  rsem).wait()
            pltpu.async_copy(d_hbm.at[r + 1], drow.at[pl.ds(_N, _N)], rsem)
            ria = _top16(drow, 0, cand_v, cand_i)
            pltpu.async_copy(f_hbm.at[ria + bbase], gbuf.at[pl.ds(0, 16)],
                             gsema)

            @pl.when(i > 0)
            def _():
                pltpu.make_async_copy(f_hbm.at[iota + bbase],
                                      gbuf.at[pl.ds(16, 16)], gsemb).wait()

                @pl.when(i > 1)
                def _():
                    pltpu.make_async_copy(orow.at[pl.ds(128, 128)],
                                          out_hbm.at[r - 3], osemb).wait()
                _reduce(gbuf, 16, orow, 128, iota)
                pltpu.async_copy(orow.at[pl.ds(128, 128)],
                                 out_hbm.at[r - 1], osemb)

            pltpu.make_async_copy(d_hbm.at[r + 1], drow.at[pl.ds(_N, _N)],
                                  rsem).wait()

            @pl.when(2 * i + 2 < _RPW)
            def _():
                pltpu.async_copy(d_hbm.at[r + 2], drow.at[pl.ds(0, _N)],
                                 rsem)
            rib = _top16(drow, _N, cand_v, cand_i)
            pltpu.async_copy(f_hbm.at[rib + bbase], gbuf.at[pl.ds(16, 16)],
                             gsemb)

            pltpu.make_async_copy(f_hbm.at[iota + bbase],
                                  gbuf.at[pl.ds(0, 16)], gsema).wait()

            @pl.when(i > 0)
            def _():
                pltpu.make_async_copy(orow.at[pl.ds(0, 128)],
                                      out_hbm.at[r - 2], osema).wait()
            _reduce(gbuf, 0, orow, 0, iota)
            pltpu.async_copy(orow.at[pl.ds(0, 128)], out_hbm.at[r], osema)
            return 0

        lax.fori_loop(0, _RPW // 2, iter_body, 0)

        pltpu.make_async_copy(f_hbm.at[iota + bbase],
                              gbuf.at[pl.ds(16, 16)], gsemb).wait()
        pltpu.make_async_copy(orow.at[pl.ds(128, 128)],
                              out_hbm.at[r0], osemb).wait()
        _reduce(gbuf, 16, orow, 128, iota)
        pltpu.async_copy(orow.at[pl.ds(128, 128)],
                         out_hbm.at[r0 + _RPW - 1], osemb)
        pltpu.make_async_copy(orow.at[pl.ds(0, 128)],
                              out_hbm.at[r0], osema).wait()
        pltpu.make_async_copy(orow.at[pl.ds(128, 128)],
                              out_hbm.at[r0], osemb).wait()

    return knn_kernel


def _mm(a, b):
    return lax.dot_general(a.astype(jnp.bfloat16), b.astype(jnp.bfloat16),
                           (((1,), (0,)), ((), ())),
                           preferred_element_type=jnp.float32)


def _dist_kernel(xr_ref, xct_ref, o_ref):
    xr = xr_ref[...]
    xct = xct_ref[0]
    sr = jnp.sum(xr * xr, axis=1, keepdims=True)
    sc = jnp.sum(xct * xct, axis=0, keepdims=True)
    o_ref[...] = (sr - 2.0 * _mm(xr, xct)) + sc


def _dist(f, ft, c):
    rb = 256
    nb = _N // rb
    return pl.pallas_call(
        _dist_kernel,
        grid=(_B, nb),
        in_specs=[
            pl.BlockSpec((rb, c), lambda b, i: (b * nb + i, 0)),
            pl.BlockSpec((1, c, _N), lambda b, i: (b, 0, 0)),
        ],
        out_specs=pl.BlockSpec((rb, _N), lambda b, i: (b * nb + i, 0)),
        out_shape=jax.ShapeDtypeStruct((_R, _N), jnp.float32),
    )(f, ft)


def _mlp1_kernel(x_ref, cov_ref, w1a_ref, w1b_ref, b1_ref, s1_ref, e1_ref,
                 w2_ref, b2_ref, s2_ref, e2_ref,
                 w3_ref, b3_ref, s3_ref, e3_ref, o_ref):
    h = (_mm(x_ref[...], w1a_ref[...]) + _mm(cov_ref[...], w1b_ref[...])
         + b1_ref[...])
    h = jnp.maximum(h * s1_ref[...] + e1_ref[...], 0.0)
    h = jnp.maximum((_mm(h, w2_ref[...]) + b2_ref[...]) * s2_ref[...]
                    + e2_ref[...], 0.0)
    h = jnp.maximum((_mm(h, w3_ref[...]) + b3_ref[...]) * s3_ref[...]
                    + e3_ref[...], 0.0)
    o_ref[...] = h


def _mlp2_kernel(p_ref, wl_ref, bl_ref, wc_ref, bc_ref, o_ref):
    h = _mm(p_ref[...], wl_ref[...]) + bl_ref[...]
    o_ref[...] = jnp.maximum(_mm(h, wc_ref[...]) + bc_ref[...], 0.0)


def _mlp3_kernel(p_ref, wl_ref, bl_ref, wc_ref, bc_ref, o_ref):
    i = pl.program_id(1)
    h = _mm(p_ref[...], wl_ref[...]) + bl_ref[...]
    g = _mm(h, wc_ref[...]) + bc_ref[...]
    m = jnp.max(g, axis=0, keepdims=True)[None]

    @pl.when(i == 0)
    def _():
        o_ref[...] = m

    @pl.when(i > 0)
    def _():
        o_ref[...] = jnp.maximum(o_ref[...], m)


def _head_kernel(x_ref, w4_ref, b4_ref, w5_ref, b5_ref, o_ref):
    h = jnp.maximum(_mm(x_ref[...], w4_ref[...]) + b4_ref[...], 0.0)
    o_ref[...] = _mm(h, w5_ref[...]) + b5_ref[...]


def _rows_spec(rb, nc):
    return pl.BlockSpec((rb, nc), lambda i: (i, 0))


def _full(shape):
    nd = len(shape)
    return pl.BlockSpec(shape, lambda *a: (0,) * nd)


@functools.lru_cache(maxsize=None)
def _sc_kernels():
    return (_make_sc_knn('cov', 16),
            _make_sc_knn('max', 64),
            _make_sc_knn('max', 128))


def kernel(x, params):
    p = params
    sc_cov, sc_pool64, sc_pool128 = _sc_kernels()

    xflat = x.reshape(_R, 3)
    xpad = jnp.concatenate(
        [xflat, jnp.zeros((_R, 125), jnp.float32)], axis=1)
    xpad_t = xpad.reshape(_B, _N, 128).transpose(0, 2, 1)

    d1 = _dist(xpad, xpad_t, 128)
    cov16 = sc_cov(d1, xpad)

    x8 = jnp.concatenate([xflat, jnp.zeros((_R, 5), jnp.float32)], axis=1)
    w1a8 = jnp.concatenate(
        [p['W1'][:3], jnp.zeros((5, 12), jnp.float32)], axis=0)
    w1b128 = jnp.concatenate(
        [p['W1'][3:], jnp.zeros((119, 12), jnp.float32)], axis=0)

    zc = jnp.zeros((64,), jnp.float32)
    w3p = jnp.concatenate([p['W3'], jnp.zeros((64, 64), jnp.float32)], axis=1)
    b3p = jnp.concatenate([p['b3'], zc])
    g3p = jnp.concatenate([p['g3'], zc])
    e3p = jnp.concatenate([p['be3'], zc])

    bnc = jnp.sqrt(jnp.float32(1.0 + 1e-3))
    s1 = p['g1'] / bnc
    s2 = p['g2'] / bnc
    s3 = g3p / bnc

    rb = 512
    nb = _R // rb
    f1 = pl.pallas_call(
        _mlp1_kernel,
        grid=(nb,),
        in_specs=[
            _rows_spec(rb, 8), _rows_spec(rb, 128),
            _full((8, 12)), _full((128, 12)),
            _full((1, 12)), _full((1, 12)), _full((1, 12)),
            _full((12, 64)), _full((1, 64)), _full((1, 64)), _full((1, 64)),
            _full((64, 128)), _full((1, 128)), _full((1, 128)),
            _full((1, 128)),
        ],
        out_specs=_rows_spec(rb, 128),
        out_shape=jax.ShapeDtypeStruct((_R, 128), jnp.float32),
    )(x8, cov16, w1a8, w1b128,
      p['b1'][None, :], s1[None, :], p['be1'][None, :],
      p['W2'], p['b2'][None, :], s2[None, :], p['be2'][None, :],
      w3p, b3p[None, :], s3[None, :], e3p[None, :])

    f1t = f1.reshape(_B, _N, 128).transpose(0, 2, 1)
    d2 = _dist(f1, f1t, 128)
    p2 = sc_pool64(d2, f1)

    wl1p = jnp.concatenate(
        [p['Wl1'], jnp.zeros((64, 64), jnp.float32)], axis=0)
    f2 = pl.pallas_call(
        _mlp2_kernel,
        grid=(nb,),
        in_specs=[
            _rows_spec(rb, 128),
            _full((128, 64)), _full((1, 64)),
            _full((64, 128)), _full((1, 128)),
        ],
        out_specs=_rows_spec(rb, 128),
        out_shape=jax.ShapeDtypeStruct((_R, 128), jnp.float32),
    )(p2, wl1p, p['bl1'][None, :], p['Wc1'], p['bc1'][None, :])

    f2t = f2.reshape(_B, _N, 128).transpose(0, 2, 1)
    d3 = _dist(f2, f2t, 128)
    p3 = sc_pool128(d3, f2)

    nbb = _N // rb
    m = pl.pallas_call(
        _mlp3_kernel,
        grid=(_B, nbb),
        in_specs=[
            pl.BlockSpec((rb, 128), lambda b, i: (b * nbb + i, 0)),
            _full((128, 128)), _full((1, 128)),
            _full((128, 1024)), _full((1, 1024)),
        ],
        out_specs=pl.BlockSpec((1, 1, 1024), lambda b, i: (b, 0, 0)),
        out_shape=jax.ShapeDtypeStruct((_B, 1, 1024), jnp.float32),
    )(p3, p['Wl2'], p['bl2'][None, :], p['Wc2'], p['bc2'][None, :])
    m = m.reshape(_B, 1024)

    out = pl.pallas_call(
        _head_kernel,
        in_specs=[
            _full((_B, 1024)),
            _full((1024, 1024)), _full((1, 1024)),
            _full((1024, 512)), _full((1, 512)),
        ],
        out_specs=_full((_B, 512)),
        out_shape=jax.ShapeDtypeStruct((_B, 512), jnp.float32),
    )(m, p['W4'], p['b4'][None, :], p['W5'], p['b5'][None, :])

    return out[:, None, :]
